# baseline (speedup 1.0000x reference)
import functools

import jax
import jax.numpy as jnp
from jax import lax
from jax.experimental import pallas as pl
from jax.experimental.pallas import tpu as pltpu
from jax.experimental.pallas import tpu_sc as plsc

N = 10000
E = 320000
D = 128
G = 64
U = 0.5
BTEMP = 0.1

NC = 2
NS = 16
NW = NC * NS

EPT = E // NW
K = 128
NCH = 162
EPC = NCH * K
EP = NS * EPC
NP = 10112
RPT = NP // NS

_f32 = jnp.float32
_i32 = jnp.int32

_sc_mesh = plsc.VectorSubcoreMesh(
    core_axis_name="c", subcore_axis_name="s", num_cores=NC, num_subcores=NS)


@functools.partial(
    pl.kernel,
    out_type=[
        jax.ShapeDtypeStruct((NW * G,), _f32),
        jax.ShapeDtypeStruct((NW * G,), _f32),
        jax.ShapeDtypeStruct((NW * N,), _f32),
        jax.ShapeDtypeStruct((NW * N,), _f32),
    ],
    mesh=_sc_mesh,
    compiler_params=pltpu.CompilerParams(needs_layout_passes=False),
    scratch_types=[
        pltpu.VMEM((N,), _i32),
        pltpu.VMEM((EPT,), _i32),
        pltpu.VMEM((EPT,), _i32),
        pltpu.VMEM((G,), _f32),
        pltpu.VMEM((G,), _f32),
        pltpu.VMEM((N,), _f32),
        pltpu.VMEM((N,), _f32),
    ],
)
def _sc_stats(batch_hbm, src_hbm, dst_hbm,
              counts_out, one_out, degtd_out, degbu_out,
              batch_v, src_v, dst_v, counts_l, one_l, degtd_l, degbu_l):
    c = lax.axis_index("c")
    s = lax.axis_index("s")
    wid = c * NS + s

    zf = jnp.zeros((16,), _f32)
    ones = jnp.ones((16,), _f32)

    @pl.loop(0, N // 16)
    def _(i):
        degtd_l[pl.ds(i * 16, 16)] = zf
        degbu_l[pl.ds(i * 16, 16)] = zf

    @pl.loop(0, G // 16)
    def _(i):
        counts_l[pl.ds(i * 16, 16)] = zf
        one_l[pl.ds(i * 16, 16)] = zf

    pltpu.sync_copy(batch_hbm, batch_v)
    pltpu.sync_copy(src_hbm.at[pl.ds(wid * EPT, EPT)], src_v)
    pltpu.sync_copy(dst_hbm.at[pl.ds(wid * EPT, EPT)], dst_v)

    iota16 = lax.iota(_i32, 16)

    @pl.loop(0, 20)
    def _(i):
        idx = wid * 320 + i * 16 + iota16
        m = idx < N
        b = plsc.load_gather(batch_v, [jnp.minimum(idx, N - 1)])
        plsc.addupdate_scatter(counts_l, [b], ones, mask=m)

    @pl.loop(0, EPT // 16)
    def _(i):
        sv = src_v[pl.ds(i * 16, 16)]
        dv = dst_v[pl.ds(i * 16, 16)]
        bs = plsc.load_gather(batch_v, [sv])
        bd = plsc.load_gather(batch_v, [dv])
        bsm1 = plsc.load_gather(batch_v, [jnp.maximum(sv - 1, 0)])
        first = (sv == 0) | (bsm1 != bs)
        valid = (bs == bd) & first
        plsc.addupdate_scatter(one_l, [bs], ones, mask=valid)
        plsc.addupdate_scatter(degtd_l, [dv], ones)
        plsc.addupdate_scatter(degbu_l, [sv], ones)

    pltpu.sync_copy(counts_l, counts_out.at[pl.ds(wid * G, G)])
    pltpu.sync_copy(one_l, one_out.at[pl.ds(wid * G, G)])
    pltpu.sync_copy(degtd_l, degtd_out.at[pl.ds(wid * N, N)])
    pltpu.sync_copy(degbu_l, degbu_out.at[pl.ds(wid * N, N)])


@functools.partial(
    pl.kernel,
    out_type=[
        jax.ShapeDtypeStruct((NP, D), _f32),
        jax.ShapeDtypeStruct((NP, D), _f32),
    ],
    mesh=_sc_mesh,
    compiler_params=pltpu.CompilerParams(needs_layout_passes=False),
    scratch_types=[
        pltpu.VMEM((K,), _i32),
        pltpu.VMEM((K,), _i32),
        pltpu.VMEM((K,), _i32),
        pltpu.VMEM((K,), _i32),
        pltpu.VMEM((K,), _i32),
        pltpu.VMEM((K,), _i32),
        pltpu.VMEM((K, D), _f32),
        pltpu.VMEM((K, D), _f32),
        pltpu.VMEM((K, D), _f32),
        pltpu.SemaphoreType.DMA,
        pltpu.SemaphoreType.DMA,
        pltpu.SemaphoreType.DMA,
        pltpu.SemaphoreType.DMA,
        pltpu.SemaphoreType.DMA,
        pltpu.SemaphoreType.DMA,
        pltpu.VMEM_SHARED((NP, D), _f32),
    ],
)
def _sc_spmm(gtd_hbm, gbu_hbm, src_hbm, dst_hbm, otd_hbm, obu_hbm,
             gi0, gi1, gi2, si0, si1, si2, buf0, buf1, buf2,
             semr0, semr1, semr2, semi0, semi1, semi2, acc):
    c = lax.axis_index("c")
    s = lax.axis_index("s")

    zf = jnp.zeros((16,), _f32)

    @pl.loop(0, 8)
    def _(r):
        for l in range(D // 16):
            buf0[r, pl.ds(l * 16, 16)] = zf

    @pl.loop(0, RPT // 8)
    def _(j):
        pltpu.sync_copy(buf0.at[pl.ds(0, 8)],
                        acc.at[pl.ds(s * RPT + j * 8, 8)])

    def _run(g_hbm, gidx_hbm, sidx_hbm, out_hbm):
        plsc.subcore_barrier()
        gis = (gi0, gi1, gi2)
        sis = (si0, si1, si2)
        bufs = (buf0, buf1, buf2)
        semrs = (semr0, semr1, semr2)
        semis = (semi0, semi1, semi2)

        def idx_start(j, p):
            base = s * EPC + j * K
            pltpu.async_copy(gidx_hbm.at[pl.ds(base, K)], gis[p], semis[p])
            pltpu.async_copy(sidx_hbm.at[pl.ds(base, K)], sis[p], semis[p])

        def idx_wait(j, p):
            base = s * EPC + j * K
            pltpu.make_async_copy(
                gidx_hbm.at[pl.ds(base, K)], gis[p], semis[p]).wait()
            pltpu.make_async_copy(
                sidx_hbm.at[pl.ds(base, K)], sis[p], semis[p]).wait()

        def half(j, p):
            pnn = (p + 2) % 3
            @pl.when(j + 2 < NCH)
            def _():
                idx_wait(j + 2, pnn)
                pltpu.async_copy(g_hbm.at[gis[pnn]], bufs[pnn], semrs[pnn])

            pltpu.make_async_copy(g_hbm.at[gis[p]], bufs[p], semrs[p]).wait()
            pltpu.sync_copy(bufs[p], acc.at[sis[p]], add=True)

            @pl.when(j + 3 < NCH)
            def _():
                idx_start(j + 3, p)

        base0 = s * EPC
        pltpu.sync_copy(gidx_hbm.at[pl.ds(base0, K)], gi0)
        pltpu.sync_copy(sidx_hbm.at[pl.ds(base0, K)], si0)
        pltpu.sync_copy(gidx_hbm.at[pl.ds(base0 + K, K)], gi1)
        pltpu.sync_copy(sidx_hbm.at[pl.ds(base0 + K, K)], si1)
        pltpu.async_copy(g_hbm.at[gi0], buf0, semr0)
        pltpu.async_copy(g_hbm.at[gi1], buf1, semr1)
        idx_start(2, 2)

        @pl.loop(0, NCH, step=3)
        def _(j):
            half(j, 0)
            half(j + 1, 1)
            half(j + 2, 2)

        plsc.subcore_barrier()
        pltpu.sync_copy(acc.at[pl.ds(s * RPT, RPT)],
                        out_hbm.at[pl.ds(s * RPT, RPT)])

    @pl.when(c == 0)
    def _():
        _run(gtd_hbm, src_hbm, dst_hbm, otd_hbm)

    @pl.when(c == 1)
    def _():
        _run(gbu_hbm, dst_hbm, src_hbm, obu_hbm)


def _prompt_mlp(xr, W1, b1, g, be, W2, b2):
    h = jnp.dot(xr, W1, preferred_element_type=_f32) + b1
    m = jnp.mean(h, axis=-1, keepdims=True)
    v = jnp.mean((h - m) * (h - m), axis=-1, keepdims=True)
    hn = (h - m) * lax.rsqrt(v + 1e-5) * g + be
    t = jnp.tanh(hn)
    return jnp.dot(t, W2, preferred_element_type=_f32) + b2


def _tc_prep_body(x_ref, batch_col_ref, counts_pT_ref, one_pT_ref,
                  degtd_pT_ref, degbu_pT_ref,
                  p1_W1, p1_b1, p1_g, p1_be, p1_W2, p1_b2,
                  p2_W1, p2_b1, p2_g, p2_be, p2_W2, p2_b2,
                  td1_W_ref, bu1_W_ref,
                  gtd1p_out, gbu1p_out, dtd_out, dbu_out):
    x = x_ref[...]
    counts = jnp.sum(counts_pT_ref[...], axis=1, keepdims=True)
    one_level = jnp.sum(one_pT_ref[...], axis=1, keepdims=True)
    degtd = jnp.sum(degtd_pT_ref[...], axis=1, keepdims=True) + 1.0
    degbu = jnp.sum(degbu_pT_ref[...], axis=1, keepdims=True) + 1.0
    dtd = lax.rsqrt(degtd)
    dbu = lax.rsqrt(degbu)

    s_ratio = one_level / jnp.maximum(counts, 1.0)
    alpha = jax.nn.sigmoid((s_ratio - U) / BTEMP)

    gi = lax.broadcasted_iota(_i32, (G, G), 0)
    gj = lax.broadcasted_iota(_i32, (G, G), 1)
    M = (gj < gi).astype(_f32)
    roots = jnp.minimum(jnp.dot(M, counts, preferred_element_type=_f32),
                        float(N - 1)).astype(_i32)

    lane_n = lax.broadcasted_iota(_i32, (G, N), 1)
    R = (lane_n == roots).astype(_f32)
    xr = jnp.dot(R, x, preferred_element_type=_f32)

    P1 = _prompt_mlp(xr, p1_W1[...], p1_b1[...], p1_g[...], p1_be[...],
                     p1_W2[...], p1_b2[...])
    P2 = _prompt_mlp(xr, p2_W1[...], p2_b1[...], p2_g[...], p2_be[...],
                     p2_W2[...], p2_b2[...])
    C1 = (1.0 - alpha) * P1 + alpha
    C2 = alpha * P2

    lane_g = lax.broadcasted_iota(_i32, (N, G), 1)
    Bmat = (lane_g == batch_col_ref[...]).astype(_f32)
    z = x * jnp.dot(Bmat, C1, preferred_element_type=_f32) + \
        jnp.dot(Bmat, C2, preferred_element_type=_f32)

    zpad = jnp.zeros((NP - N, D), _f32)
    gtd1p_out[...] = jnp.concatenate(
        [dtd * jnp.dot(z, td1_W_ref[...], preferred_element_type=_f32), zpad],
        axis=0)
    gbu1p_out[...] = jnp.concatenate(
        [dbu * jnp.dot(z, bu1_W_ref[...], preferred_element_type=_f32), zpad],
        axis=0)
    dtd_out[...] = dtd
    dbu_out[...] = dbu


_tc_prep = pl.pallas_call(
    _tc_prep_body,
    out_shape=[
        jax.ShapeDtypeStruct((NP, D), _f32),
        jax.ShapeDtypeStruct((NP, D), _f32),
        jax.ShapeDtypeStruct((N, 1), _f32),
        jax.ShapeDtypeStruct((N, 1), _f32),
    ],
)


def _tc_mid_body(acc_td_ref, acc_bu_ref, gtd_ref, gbu_ref, dtd_ref, dbu_ref,
                 td_b_ref, bu_b_ref, td_W_ref, bu_W_ref,
                 gtd2_out, gbu2_out):
    dtd = dtd_ref[...]
    dbu = dbu_ref[...]
    acc_td = acc_td_ref[...][:N]
    acc_bu = acc_bu_ref[...][:N]
    gtd = gtd_ref[...][:N]
    gbu = gbu_ref[...][:N]
    h_td = jnp.maximum(dtd * (acc_td + gtd) + td_b_ref[...], 0.0)
    h_bu = jnp.maximum(dbu * (acc_bu + gbu) + bu_b_ref[...], 0.0)
    zpad = jnp.zeros((NP - N, D), _f32)
    gtd2_out[...] = jnp.concatenate(
        [dtd * jnp.dot(h_td, td_W_ref[...], preferred_element_type=_f32),
         zpad], axis=0)
    gbu2_out[...] = jnp.concatenate(
        [dbu * jnp.dot(h_bu, bu_W_ref[...], preferred_element_type=_f32),
         zpad], axis=0)


_tc_mid = pl.pallas_call(
    _tc_mid_body,
    out_shape=[
        jax.ShapeDtypeStruct((NP, D), _f32),
        jax.ShapeDtypeStruct((NP, D), _f32),
    ],
)


def _tc_final_body(acc_td_ref, acc_bu_ref, gtd_ref, gbu_ref, dtd_ref, dbu_ref,
                   td_b_ref, bu_b_ref, batch_row_ref,
                   ph1_W_ref, ph1_b_ref, ph2_W_ref, ph2_b_ref, out_ref):
    h_td = (dtd_ref[...] * (acc_td_ref[...][:N] + gtd_ref[...][:N])
            + td_b_ref[...])
    h_bu = (dbu_ref[...] * (acc_bu_ref[...][:N] + gbu_ref[...][:N])
            + bu_b_ref[...])
    sub_g = lax.broadcasted_iota(_i32, (G, N), 0)
    BmatT = (sub_g == batch_row_ref[...]).astype(_f32)
    TD_x = jnp.dot(BmatT, h_td, preferred_element_type=_f32)
    BU_x = jnp.dot(BmatT, h_bu, preferred_element_type=_f32)
    hcat = jnp.concatenate([BU_x, TD_x], axis=1)
    h1 = jnp.maximum(jnp.dot(hcat, ph1_W_ref[...],
                             preferred_element_type=_f32) + ph1_b_ref[...],
                     0.0)
    out_ref[...] = jnp.dot(h1, ph2_W_ref[...],
                           preferred_element_type=_f32) + ph2_b_ref[...]


_tc_final = pl.pallas_call(
    _tc_final_body,
    out_shape=jax.ShapeDtypeStruct((G, D), _f32),
)


@jax.jit
def kernel(x, p1_W1, p1_b1, p1_g, p1_be, p1_W2, p1_b2,
           p2_W1, p2_b1, p2_g, p2_be, p2_W2, p2_b2,
           td1_W, td1_b, td2_W, td2_b,
           bu1_W, bu1_b, bu2_W, bu2_b,
           ph1_W, ph1_b, ph2_W, ph2_b,
           edge_index, batch):
    pad = N + (jnp.arange(EP - E, dtype=_i32) % (NP - N))
    src = jnp.concatenate([edge_index[0], pad])
    dst = jnp.concatenate([edge_index[1], pad])
    batch_col = batch.reshape(N, 1)
    batch_row = batch.reshape(1, N)

    counts_p, one_p, degtd_p, degbu_p = _sc_stats(
        batch, edge_index[0], edge_index[1])

    gtd1p, gbu1p, dtd, dbu = _tc_prep(
        x, batch_col,
        counts_p.reshape(NW, G).T, one_p.reshape(NW, G).T,
        degtd_p.reshape(NW, N).T, degbu_p.reshape(NW, N).T,
        p1_W1, p1_b1.reshape(1, D), p1_g.reshape(1, D), p1_be.reshape(1, D),
        p1_W2, p1_b2.reshape(1, D),
        p2_W1, p2_b1.reshape(1, D), p2_g.reshape(1, D), p2_be.reshape(1, D),
        p2_W2, p2_b2.reshape(1, D),
        td1_W, bu1_W)

    acc_td1, acc_bu1 = _sc_spmm(gtd1p, gbu1p, src, dst)

    gtd2p, gbu2p = _tc_mid(
        acc_td1, acc_bu1, gtd1p, gbu1p, dtd, dbu,
        td1_b.reshape(1, D), bu1_b.reshape(1, D), td2_W, bu2_W)

    acc_td2, acc_bu2 = _sc_spmm(gtd2p, gbu2p, src, dst)

    out = _tc_final(
        acc_td2, acc_bu2, gtd2p, gbu2p, dtd, dbu,
        td2_b.reshape(1, D), bu2_b.reshape(1, D), batch_row,
        ph1_W, ph1_b.reshape(1, 2 * D), ph2_W, ph2_b.reshape(1, D))
    return out


# speedup vs baseline: 1.0019x; 1.0019x over previous
import functools

import jax
import jax.numpy as jnp
from jax import lax
from jax.experimental import pallas as pl
from jax.experimental.pallas import tpu as pltpu
from jax.experimental.pallas import tpu_sc as plsc

N = 10000
E = 320000
D = 128
G = 64
U = 0.5
BTEMP = 0.1

NC = 2
NS = 16
NW = NC * NS

EPT = E // NW
K = 112
NCH = 180
EPC = NCH * K
EP = NS * EPC
NP = 10240
RPT = NP // NS

_f32 = jnp.float32
_i32 = jnp.int32

_sc_mesh = plsc.VectorSubcoreMesh(
    core_axis_name="c", subcore_axis_name="s", num_cores=NC, num_subcores=NS)


@functools.partial(
    pl.kernel,
    out_type=[
        jax.ShapeDtypeStruct((NW * G,), _f32),
        jax.ShapeDtypeStruct((NW * G,), _f32),
        jax.ShapeDtypeStruct((NW * N,), _f32),
        jax.ShapeDtypeStruct((NW * N,), _f32),
    ],
    mesh=_sc_mesh,
    compiler_params=pltpu.CompilerParams(needs_layout_passes=False),
    scratch_types=[
        pltpu.VMEM((N,), _i32),
        pltpu.VMEM((EPT,), _i32),
        pltpu.VMEM((EPT,), _i32),
        pltpu.VMEM((G,), _f32),
        pltpu.VMEM((G,), _f32),
        pltpu.VMEM((N,), _f32),
        pltpu.VMEM((N,), _f32),
    ],
)
def _sc_stats(batch_hbm, src_hbm, dst_hbm,
              counts_out, one_out, degtd_out, degbu_out,
              batch_v, src_v, dst_v, counts_l, one_l, degtd_l, degbu_l):
    c = lax.axis_index("c")
    s = lax.axis_index("s")
    wid = c * NS + s

    zf = jnp.zeros((16,), _f32)
    ones = jnp.ones((16,), _f32)

    @pl.loop(0, N // 16)
    def _(i):
        degtd_l[pl.ds(i * 16, 16)] = zf
        degbu_l[pl.ds(i * 16, 16)] = zf

    @pl.loop(0, G // 16)
    def _(i):
        counts_l[pl.ds(i * 16, 16)] = zf
        one_l[pl.ds(i * 16, 16)] = zf

    pltpu.sync_copy(batch_hbm, batch_v)
    pltpu.sync_copy(src_hbm.at[pl.ds(wid * EPT, EPT)], src_v)
    pltpu.sync_copy(dst_hbm.at[pl.ds(wid * EPT, EPT)], dst_v)

    iota16 = lax.iota(_i32, 16)

    @pl.loop(0, 20)
    def _(i):
        idx = wid * 320 + i * 16 + iota16
        m = idx < N
        b = plsc.load_gather(batch_v, [jnp.minimum(idx, N - 1)])
        plsc.addupdate_scatter(counts_l, [b], ones, mask=m)

    @pl.loop(0, EPT // 16)
    def _(i):
        sv = src_v[pl.ds(i * 16, 16)]
        dv = dst_v[pl.ds(i * 16, 16)]
        bs = plsc.load_gather(batch_v, [sv])
        bd = plsc.load_gather(batch_v, [dv])
        bsm1 = plsc.load_gather(batch_v, [jnp.maximum(sv - 1, 0)])
        first = (sv == 0) | (bsm1 != bs)
        valid = (bs == bd) & first
        plsc.addupdate_scatter(one_l, [bs], ones, mask=valid)
        plsc.addupdate_scatter(degtd_l, [dv], ones)
        plsc.addupdate_scatter(degbu_l, [sv], ones)

    pltpu.sync_copy(counts_l, counts_out.at[pl.ds(wid * G, G)])
    pltpu.sync_copy(one_l, one_out.at[pl.ds(wid * G, G)])
    pltpu.sync_copy(degtd_l, degtd_out.at[pl.ds(wid * N, N)])
    pltpu.sync_copy(degbu_l, degbu_out.at[pl.ds(wid * N, N)])


@functools.partial(
    pl.kernel,
    out_type=[
        jax.ShapeDtypeStruct((NP, D), _f32),
        jax.ShapeDtypeStruct((NP, D), _f32),
    ],
    mesh=_sc_mesh,
    compiler_params=pltpu.CompilerParams(needs_layout_passes=False),
    scratch_types=[
        pltpu.VMEM((K,), _i32),
        pltpu.VMEM((K,), _i32),
        pltpu.VMEM((K,), _i32),
        pltpu.VMEM((K,), _i32),
        pltpu.VMEM((K,), _i32),
        pltpu.VMEM((K,), _i32),
        pltpu.VMEM((K, D), _f32),
        pltpu.VMEM((K, D), _f32),
        pltpu.VMEM((K, D), _f32),
        pltpu.SemaphoreType.DMA,
        pltpu.SemaphoreType.DMA,
        pltpu.SemaphoreType.DMA,
        pltpu.SemaphoreType.DMA,
        pltpu.SemaphoreType.DMA,
        pltpu.SemaphoreType.DMA,
        pltpu.VMEM_SHARED((NP, D), _f32),
    ],
)
def _sc_spmm(gtd_hbm, gbu_hbm, src_hbm, dst_hbm, otd_hbm, obu_hbm,
             gi0, gi1, gi2, si0, si1, si2, buf0, buf1, buf2,
             semr0, semr1, semr2, semi0, semi1, semi2, acc):
    c = lax.axis_index("c")
    s = lax.axis_index("s")

    zf = jnp.zeros((16,), _f32)

    @pl.loop(0, 80)
    def _(r):
        for l in range(D // 16):
            buf0[r, pl.ds(l * 16, 16)] = zf

    @pl.loop(0, RPT // 80)
    def _(j):
        pltpu.sync_copy(buf0.at[pl.ds(0, 80)],
                        acc.at[pl.ds(s * RPT + j * 80, 80)])

    def _run(g_hbm, gidx_hbm, sidx_hbm, out_hbm):
        plsc.subcore_barrier()
        gis = (gi0, gi1, gi2)
        sis = (si0, si1, si2)
        bufs = (buf0, buf1, buf2)
        semrs = (semr0, semr1, semr2)
        semis = (semi0, semi1, semi2)

        def idx_start(j, p):
            base = s * EPC + j * K
            pltpu.async_copy(gidx_hbm.at[pl.ds(base, K)], gis[p], semis[p])
            pltpu.async_copy(sidx_hbm.at[pl.ds(base, K)], sis[p], semis[p])

        def idx_wait(j, p):
            base = s * EPC + j * K
            pltpu.make_async_copy(
                gidx_hbm.at[pl.ds(base, K)], gis[p], semis[p]).wait()
            pltpu.make_async_copy(
                sidx_hbm.at[pl.ds(base, K)], sis[p], semis[p]).wait()

        def half(j, p):
            pnn = (p + 2) % 3
            @pl.when(j + 2 < NCH)
            def _():
                idx_wait(j + 2, pnn)
                pltpu.async_copy(g_hbm.at[gis[pnn]], bufs[pnn], semrs[pnn])

            pltpu.make_async_copy(g_hbm.at[gis[p]], bufs[p], semrs[p]).wait()
            pltpu.sync_copy(bufs[p], acc.at[sis[p]], add=True)

            @pl.when(j + 3 < NCH)
            def _():
                idx_start(j + 3, p)

        base0 = s * EPC
        pltpu.sync_copy(gidx_hbm.at[pl.ds(base0, K)], gi0)
        pltpu.sync_copy(sidx_hbm.at[pl.ds(base0, K)], si0)
        pltpu.sync_copy(gidx_hbm.at[pl.ds(base0 + K, K)], gi1)
        pltpu.sync_copy(sidx_hbm.at[pl.ds(base0 + K, K)], si1)
        pltpu.async_copy(g_hbm.at[gi0], buf0, semr0)
        pltpu.async_copy(g_hbm.at[gi1], buf1, semr1)
        idx_start(2, 2)

        @pl.loop(0, NCH, step=3)
        def _(j):
            half(j, 0)
            half(j + 1, 1)
            half(j + 2, 2)

        plsc.subcore_barrier()
        pltpu.sync_copy(acc.at[pl.ds(s * RPT, RPT)],
                        out_hbm.at[pl.ds(s * RPT, RPT)])

    @pl.when(c == 0)
    def _():
        _run(gtd_hbm, src_hbm, dst_hbm, otd_hbm)

    @pl.when(c == 1)
    def _():
        _run(gbu_hbm, dst_hbm, src_hbm, obu_hbm)


def _prompt_mlp(xr, W1, b1, g, be, W2, b2):
    h = jnp.dot(xr, W1, preferred_element_type=_f32) + b1
    m = jnp.mean(h, axis=-1, keepdims=True)
    v = jnp.mean((h - m) * (h - m), axis=-1, keepdims=True)
    hn = (h - m) * lax.rsqrt(v + 1e-5) * g + be
    t = jnp.tanh(hn)
    return jnp.dot(t, W2, preferred_element_type=_f32) + b2


def _tc_prep_body(x_ref, batch_col_ref, counts_pT_ref, one_pT_ref,
                  degtd_pT_ref, degbu_pT_ref,
                  p1_W1, p1_b1, p1_g, p1_be, p1_W2, p1_b2,
                  p2_W1, p2_b1, p2_g, p2_be, p2_W2, p2_b2,
                  td1_W_ref, bu1_W_ref,
                  gtd1p_out, gbu1p_out, dtd_out, dbu_out):
    x = x_ref[...]
    counts = jnp.sum(counts_pT_ref[...], axis=1, keepdims=True)
    one_level = jnp.sum(one_pT_ref[...], axis=1, keepdims=True)
    degtd = jnp.sum(degtd_pT_ref[...], axis=1, keepdims=True) + 1.0
    degbu = jnp.sum(degbu_pT_ref[...], axis=1, keepdims=True) + 1.0
    dtd = lax.rsqrt(degtd)
    dbu = lax.rsqrt(degbu)

    s_ratio = one_level / jnp.maximum(counts, 1.0)
    alpha = jax.nn.sigmoid((s_ratio - U) / BTEMP)

    gi = lax.broadcasted_iota(_i32, (G, G), 0)
    gj = lax.broadcasted_iota(_i32, (G, G), 1)
    M = (gj < gi).astype(_f32)
    roots = jnp.minimum(jnp.dot(M, counts, preferred_element_type=_f32),
                        float(N - 1)).astype(_i32)

    lane_n = lax.broadcasted_iota(_i32, (G, N), 1)
    R = (lane_n == roots).astype(_f32)
    xr = jnp.dot(R, x, preferred_element_type=_f32)

    P1 = _prompt_mlp(xr, p1_W1[...], p1_b1[...], p1_g[...], p1_be[...],
                     p1_W2[...], p1_b2[...])
    P2 = _prompt_mlp(xr, p2_W1[...], p2_b1[...], p2_g[...], p2_be[...],
                     p2_W2[...], p2_b2[...])
    C1 = (1.0 - alpha) * P1 + alpha
    C2 = alpha * P2

    lane_g = lax.broadcasted_iota(_i32, (N, G), 1)
    Bmat = (lane_g == batch_col_ref[...]).astype(_f32)
    z = x * jnp.dot(Bmat, C1, preferred_element_type=_f32) + \
        jnp.dot(Bmat, C2, preferred_element_type=_f32)

    zpad = jnp.zeros((NP - N, D), _f32)
    gtd1p_out[...] = jnp.concatenate(
        [dtd * jnp.dot(z, td1_W_ref[...], preferred_element_type=_f32), zpad],
        axis=0)
    gbu1p_out[...] = jnp.concatenate(
        [dbu * jnp.dot(z, bu1_W_ref[...], preferred_element_type=_f32), zpad],
        axis=0)
    dtd_out[...] = dtd
    dbu_out[...] = dbu


_tc_prep = pl.pallas_call(
    _tc_prep_body,
    out_shape=[
        jax.ShapeDtypeStruct((NP, D), _f32),
        jax.ShapeDtypeStruct((NP, D), _f32),
        jax.ShapeDtypeStruct((N, 1), _f32),
        jax.ShapeDtypeStruct((N, 1), _f32),
    ],
)


def _tc_mid_body(acc_td_ref, acc_bu_ref, gtd_ref, gbu_ref, dtd_ref, dbu_ref,
                 td_b_ref, bu_b_ref, td_W_ref, bu_W_ref,
                 gtd2_out, gbu2_out):
    dtd = dtd_ref[...]
    dbu = dbu_ref[...]
    acc_td = acc_td_ref[...][:N]
    acc_bu = acc_bu_ref[...][:N]
    gtd = gtd_ref[...][:N]
    gbu = gbu_ref[...][:N]
    h_td = jnp.maximum(dtd * (acc_td + gtd) + td_b_ref[...], 0.0)
    h_bu = jnp.maximum(dbu * (acc_bu + gbu) + bu_b_ref[...], 0.0)
    zpad = jnp.zeros((NP - N, D), _f32)
    gtd2_out[...] = jnp.concatenate(
        [dtd * jnp.dot(h_td, td_W_ref[...], preferred_element_type=_f32),
         zpad], axis=0)
    gbu2_out[...] = jnp.concatenate(
        [dbu * jnp.dot(h_bu, bu_W_ref[...], preferred_element_type=_f32),
         zpad], axis=0)


_tc_mid = pl.pallas_call(
    _tc_mid_body,
    out_shape=[
        jax.ShapeDtypeStruct((NP, D), _f32),
        jax.ShapeDtypeStruct((NP, D), _f32),
    ],
)


def _tc_final_body(acc_td_ref, acc_bu_ref, gtd_ref, gbu_ref, dtd_ref, dbu_ref,
                   td_b_ref, bu_b_ref, batch_row_ref,
                   ph1_W_ref, ph1_b_ref, ph2_W_ref, ph2_b_ref, out_ref):
    h_td = (dtd_ref[...] * (acc_td_ref[...][:N] + gtd_ref[...][:N])
            + td_b_ref[...])
    h_bu = (dbu_ref[...] * (acc_bu_ref[...][:N] + gbu_ref[...][:N])
            + bu_b_ref[...])
    sub_g = lax.broadcasted_iota(_i32, (G, N), 0)
    BmatT = (sub_g == batch_row_ref[...]).astype(_f32)
    TD_x = jnp.dot(BmatT, h_td, preferred_element_type=_f32)
    BU_x = jnp.dot(BmatT, h_bu, preferred_element_type=_f32)
    hcat = jnp.concatenate([BU_x, TD_x], axis=1)
    h1 = jnp.maximum(jnp.dot(hcat, ph1_W_ref[...],
                             preferred_element_type=_f32) + ph1_b_ref[...],
                     0.0)
    out_ref[...] = jnp.dot(h1, ph2_W_ref[...],
                           preferred_element_type=_f32) + ph2_b_ref[...]


_tc_final = pl.pallas_call(
    _tc_final_body,
    out_shape=jax.ShapeDtypeStruct((G, D), _f32),
)


@jax.jit
def kernel(x, p1_W1, p1_b1, p1_g, p1_be, p1_W2, p1_b2,
           p2_W1, p2_b1, p2_g, p2_be, p2_W2, p2_b2,
           td1_W, td1_b, td2_W, td2_b,
           bu1_W, bu1_b, bu2_W, bu2_b,
           ph1_W, ph1_b, ph2_W, ph2_b,
           edge_index, batch):
    pad = N + (jnp.arange(EP - E, dtype=_i32) % (NP - N))
    src = jnp.concatenate([edge_index[0], pad])
    dst = jnp.concatenate([edge_index[1], pad])
    batch_col = batch.reshape(N, 1)
    batch_row = batch.reshape(1, N)

    counts_p, one_p, degtd_p, degbu_p = _sc_stats(
        batch, edge_index[0], edge_index[1])

    gtd1p, gbu1p, dtd, dbu = _tc_prep(
        x, batch_col,
        counts_p.reshape(NW, G).T, one_p.reshape(NW, G).T,
        degtd_p.reshape(NW, N).T, degbu_p.reshape(NW, N).T,
        p1_W1, p1_b1.reshape(1, D), p1_g.reshape(1, D), p1_be.reshape(1, D),
        p1_W2, p1_b2.reshape(1, D),
        p2_W1, p2_b1.reshape(1, D), p2_g.reshape(1, D), p2_be.reshape(1, D),
        p2_W2, p2_b2.reshape(1, D),
        td1_W, bu1_W)

    acc_td1, acc_bu1 = _sc_spmm(gtd1p, gbu1p, src, dst)

    gtd2p, gbu2p = _tc_mid(
        acc_td1, acc_bu1, gtd1p, gbu1p, dtd, dbu,
        td1_b.reshape(1, D), bu1_b.reshape(1, D), td2_W, bu2_W)

    acc_td2, acc_bu2 = _sc_spmm(gtd2p, gbu2p, src, dst)

    out = _tc_final(
        acc_td2, acc_bu2, gtd2p, gbu2p, dtd, dbu,
        td2_b.reshape(1, D), bu2_b.reshape(1, D), batch_row,
        ph1_W, ph1_b.reshape(1, 2 * D), ph2_W, ph2_b.reshape(1, D))
    return out
